# baseline (speedup 1.0000x reference)
import functools

import jax
import jax.numpy as jnp
from jax import lax
from jax.experimental import pallas as pl
from jax.experimental.pallas import tpu as pltpu
from jax.experimental.pallas import tpu_sc as plsc

VOCAB = 100000
EMBED_DIM = 16
BATCH = 1024
CTX = 20

NC = 2
NS = 16
NW = NC * NS
B_PER_W = BATCH // NW
IDX_PER_W = B_PER_W * CTX
IDX_CHUNK = 128
N_CHUNKS = IDX_PER_W // IDX_CHUNK

VT = 2048


def _pool_sc(idx3, table):
    mesh = plsc.VectorSubcoreMesh(core_axis_name="c", subcore_axis_name="s")

    @functools.partial(
        pl.kernel,
        mesh=mesh,
        out_type=jax.ShapeDtypeStruct((BATCH, EMBED_DIM), jnp.float32),
        scratch_types=[
            pltpu.VMEM((N_CHUNKS, IDX_CHUNK), jnp.int32),
            pltpu.VMEM((IDX_PER_W, EMBED_DIM), jnp.float32),
            pltpu.VMEM((B_PER_W, EMBED_DIM), jnp.float32),
            pltpu.SemaphoreType.DMA,
        ],
        compiler_params=pltpu.CompilerParams(use_tc_tiling_on_sc=False),
    )
    def k(idx_hbm, table_hbm, out_hbm, idx_v, rows_v, pooled_v, sem):
        wid = lax.axis_index("s") * NC + lax.axis_index("c")
        pltpu.sync_copy(idx_hbm.at[wid], idx_v)
        copies = [
            pltpu.async_copy(
                table_hbm.at[idx_v.at[j]],
                rows_v.at[pl.ds(j * IDX_CHUNK, IDX_CHUNK)],
                sem,
            )
            for j in range(N_CHUNKS)
        ]
        for c in copies:
            c.wait()

        def body(b, carry):
            r0 = b * CTX
            acc = rows_v[r0]
            for j in range(1, CTX):
                acc = acc + rows_v[r0 + j]
            pooled_v[b] = acc
            return carry

        lax.fori_loop(0, B_PER_W, body, 0)
        pltpu.sync_copy(pooled_v, out_hbm.at[pl.ds(wid * B_PER_W, B_PER_W)])

    return k(idx3, table)


BT = 16
NBUF = 6
NSTEP = BATCH // BT


def _mm_body(x_ref, wt_ref, b_ref, o_hbm, scratch, sems):
    i = pl.program_id(0)
    slot = lax.rem(i, NBUF)

    @pl.when(i >= NBUF)
    def _wait_prev():
        j = i - NBUF
        pltpu.make_async_copy(
            scratch.at[slot], o_hbm.at[pl.ds(j * BT, BT)], sems.at[slot]
        ).wait()

    scratch[slot] = lax.dot_general(
        x_ref[...], wt_ref[...],
        dimension_numbers=(((1,), (0,)), ((), ())),
        preferred_element_type=jnp.float32,
    ) + b_ref[...]
    pltpu.make_async_copy(
        scratch.at[slot], o_hbm.at[pl.ds(i * BT, BT)], sems.at[slot]
    ).start()

    @pl.when(i == NSTEP - 1)
    def _drain():
        for k in range(NBUF):
            j = i - (NBUF - 1) + k
            pltpu.make_async_copy(
                scratch.at[lax.rem(j, NBUF)],
                o_hbm.at[pl.ds(j * BT, BT)],
                sems.at[lax.rem(j, NBUF)],
            ).wait()


def _project_tc(x, Wt, b2):
    return pl.pallas_call(
        _mm_body,
        grid=(NSTEP,),
        in_specs=[
            pl.BlockSpec((BT, EMBED_DIM), lambda i: (i, 0)),
            pl.BlockSpec((EMBED_DIM, VOCAB), lambda i: (0, 0)),
            pl.BlockSpec((1, VOCAB), lambda i: (0, 0)),
        ],
        out_specs=pl.BlockSpec(memory_space=pl.ANY),
        out_shape=jax.ShapeDtypeStruct((BATCH, VOCAB), jnp.float32),
        scratch_shapes=[
            pltpu.VMEM((NBUF, BT, VOCAB), jnp.float32),
            pltpu.SemaphoreType.DMA((NBUF,)),
        ],
        compiler_params=pltpu.CompilerParams(
            vmem_limit_bytes=100 * 1024 * 1024,
        ),
    )(x, Wt, b2)


def kernel(inputs, embed_table, W, b):
    pooled = jnp.sum(jnp.take(embed_table, inputs, axis=0), axis=1)
    return _project_tc(pooled, W.T, b.reshape(1, VOCAB))


# speedup vs baseline: 2.8246x; 2.8246x over previous
import functools

import jax
import jax.numpy as jnp
from jax import lax
from jax.experimental import pallas as pl
from jax.experimental.pallas import tpu as pltpu
from jax.experimental.pallas import tpu_sc as plsc

VOCAB = 100000
EMBED_DIM = 16
BATCH = 1024
CTX = 20

VT = 2048
NBUF = 4
NSTEP = (VOCAB + VT - 1) // VT
LAST = VOCAB - (NSTEP - 1) * VT


def _mm_body(wt_ref, b_ref, xt_ref, o_hbm, scratch, sems):
    i = pl.program_id(0)
    slot = lax.rem(i, NBUF)

    @pl.when(i >= NBUF)
    def _wait_prev():
        j = i - NBUF
        pltpu.make_async_copy(
            scratch.at[slot], o_hbm.at[pl.ds(j * VT, VT)], sems.at[slot]
        ).wait()

    lhs = jnp.concatenate([wt_ref[...], b_ref[...]], axis=0)
    scratch[slot] = lax.dot_general(
        lhs, xt_ref[...],
        dimension_numbers=(((0,), (0,)), ((), ())),
        preferred_element_type=jnp.float32,
    )

    @pl.when(i < NSTEP - 1)
    def _start_full():
        pltpu.make_async_copy(
            scratch.at[slot], o_hbm.at[pl.ds(i * VT, VT)], sems.at[slot]
        ).start()

    @pl.when(i == NSTEP - 1)
    def _start_last():
        pltpu.make_async_copy(
            scratch.at[slot, pl.ds(0, LAST)],
            o_hbm.at[pl.ds(i * VT, LAST)],
            sems.at[slot],
        ).start()

    @pl.when(i == NSTEP - 1)
    def _drain():
        for k in range(NBUF - 1):
            j = i - (NBUF - 1) + k
            pltpu.make_async_copy(
                scratch.at[lax.rem(j, NBUF)],
                o_hbm.at[pl.ds(j * VT, VT)],
                sems.at[lax.rem(j, NBUF)],
            ).wait()
        pltpu.make_async_copy(
            scratch.at[slot, pl.ds(0, LAST)],
            o_hbm.at[pl.ds(i * VT, LAST)],
            sems.at[slot],
        ).wait()


def _project_tc(Wt, b2, xt_aug):
    out_t = pl.pallas_call(
        _mm_body,
        grid=(NSTEP,),
        in_specs=[
            pl.BlockSpec((EMBED_DIM, VT), lambda i: (0, i)),
            pl.BlockSpec((1, VT), lambda i: (0, i)),
            pl.BlockSpec((EMBED_DIM + 1, BATCH), lambda i: (0, 0)),
        ],
        out_specs=pl.BlockSpec(memory_space=pl.ANY),
        out_shape=jax.ShapeDtypeStruct((VOCAB, BATCH), jnp.float32),
        scratch_shapes=[
            pltpu.VMEM((NBUF, VT, BATCH), jnp.float32),
            pltpu.SemaphoreType.DMA((NBUF,)),
        ],
        compiler_params=pltpu.CompilerParams(
            vmem_limit_bytes=100 * 1024 * 1024,
        ),
    )(Wt, b2, xt_aug)
    return out_t


def kernel(inputs, embed_table, W, b):
    pooled = jnp.sum(jnp.take(embed_table, inputs, axis=0), axis=1)
    xt_aug = jnp.concatenate(
        [pooled.T, jnp.ones((1, BATCH), jnp.float32)], axis=0
    )
    out_t = _project_tc(W.T, b.reshape(1, VOCAB), xt_aug)
    return out_t.T
